# Initial kernel scaffold; baseline (speedup 1.0000x reference)
import jax
import jax.numpy as jnp
from jax.experimental import pallas as pl


def kernel(nodes, x, edge_index, W):
    raise NotImplementedError("write your pallas kernel here")


# speedup vs baseline: 2.5822x; 2.5822x over previous
import functools

import jax
import jax.numpy as jnp
from jax import lax
from jax.experimental import pallas as pl
from jax.experimental.pallas import tpu as pltpu
from jax.experimental.pallas import tpu_sc as plsc

N_NODES = 10000
N_EDGES = 160000
D_FEAT = 256
HALF = 128

NC = 2
NS = 16
NW = NC * NS

E_PAD = 163840
EPT = E_PAD // NS
FULL_CHUNKS = EPT // 128
DEG_CHUNKS = FULL_CHUNKS // NC
N_PAD = 10112
ROW0_STRIDE = 640
D_PAD = 10240

B_PAD = 10240
BPT = B_PAD // NW


def _seg_sum_sc(xr, idx2f, dstp, zf):
    mesh = plsc.VectorSubcoreMesh(core_axis_name="c", subcore_axis_name="s")

    @functools.partial(
        pl.kernel,
        mesh=mesh,
        out_type=jax.ShapeDtypeStruct((NC * N_PAD, HALF), jnp.float32),
        scratch_types=[
            pltpu.VMEM((128,), jnp.int32),
            pltpu.VMEM((128,), jnp.int32),
            pltpu.VMEM((128, HALF), jnp.float32),
            pltpu.VMEM_SHARED((N_PAD, HALF), jnp.float32),
            pltpu.SemaphoreType.DMA,
        ],
    )
    def k(xr_h, idx2_h, dst_h, zf_h,
          neigh_o,
          idx_v, dst_v, rows_v, acc, sem):
        c = lax.axis_index("c")
        s = lax.axis_index("s")
        row0 = pl.multiple_of(s * ROW0_STRIDE, 128)
        nck = jnp.where(s == NS - 1, 4, 5)

        pltpu.sync_copy(zf_h, rows_v)

        @pl.loop(0, nck)
        def _zero(kk):
            roff = pl.multiple_of(row0 + kk * 128, 128)
            pltpu.sync_copy(rows_v, acc.at[pl.ds(roff, 128)])

        plsc.subcore_barrier()

        ebase = pl.multiple_of(c * E_PAD + s * EPT, 128)
        dbase = pl.multiple_of(s * EPT, 128)
        @pl.loop(0, FULL_CHUNKS)
        def _feat(j):
            off = pl.multiple_of(ebase + j * 128, 128)
            eoff = pl.multiple_of(dbase + j * 128, 128)
            pltpu.sync_copy(idx2_h.at[pl.ds(off, 128)], idx_v)
            pltpu.sync_copy(dst_h.at[pl.ds(eoff, 128)], dst_v)
            pltpu.async_copy(xr_h.at[idx_v], rows_v, sem).wait()
            pltpu.sync_copy(rows_v, acc.at[dst_v], add=True)


        plsc.subcore_barrier()

        orow0 = pl.multiple_of(c * N_PAD + s * ROW0_STRIDE, 128)

        @pl.loop(0, nck)
        def _wb(kk):
            roff = pl.multiple_of(row0 + kk * 128, 128)
            ooff = pl.multiple_of(orow0 + kk * 128, 128)
            pltpu.sync_copy(acc.at[pl.ds(roff, 128)], rows_v)
            pltpu.sync_copy(rows_v, neigh_o.at[pl.ds(ooff, 128)])

    return k(xr, idx2f, dstp, zf)


def _deg_sc(dstp, zd, ones_f):
    mesh = plsc.VectorSubcoreMesh(core_axis_name="c", subcore_axis_name="s")

    @functools.partial(
        pl.kernel,
        mesh=mesh,
        out_type=jax.ShapeDtypeStruct((NC * N_PAD, 128), jnp.float32),
        scratch_types=[
            pltpu.VMEM((128,), jnp.int32),
            pltpu.VMEM((128, 128), jnp.float32),
            pltpu.VMEM((128, 128), jnp.float32),
            pltpu.VMEM_SHARED((N_PAD, 128), jnp.float32),
        ],
    )
    def k(dst_h, zd_h, ones_fh, degs_o, dst_v, ones_v, small_v, dacc):
        c = lax.axis_index("c")
        s = lax.axis_index("s")
        row0 = pl.multiple_of(s * ROW0_STRIDE, 128)
        nck = jnp.where(s == NS - 1, 4, 5)

        pltpu.sync_copy(zd_h, small_v)
        pltpu.sync_copy(ones_fh, ones_v)

        @pl.loop(0, nck)
        def _zero(kk):
            roff = pl.multiple_of(row0 + kk * 128, 128)
            pltpu.sync_copy(small_v, dacc.at[pl.ds(roff, 128)])

        plsc.subcore_barrier()

        gbase = pl.multiple_of((c * NS + s) * (E_PAD // NW), 128)

        @pl.loop(0, FULL_CHUNKS // NC)
        def _deg(j):
            off = pl.multiple_of(gbase + j * 128, 128)
            pltpu.sync_copy(dst_h.at[pl.ds(off, 128)], dst_v)
            pltpu.sync_copy(ones_v, dacc.at[dst_v], add=True)

        plsc.subcore_barrier()

        orow0 = pl.multiple_of(c * N_PAD + s * ROW0_STRIDE, 128)

        @pl.loop(0, nck)
        def _wb(kk):
            roff = pl.multiple_of(row0 + kk * 128, 128)
            ooff = pl.multiple_of(orow0 + kk * 128, 128)
            pltpu.sync_copy(dacc.at[pl.ds(roff, 128)], small_v)
            pltpu.sync_copy(small_v, degs_o.at[pl.ds(ooff, 128)])

    return k(dstp, zd, ones_f)


def _gather_sc(h, nodesp):
    mesh = plsc.VectorSubcoreMesh(core_axis_name="c", subcore_axis_name="s")

    @functools.partial(
        pl.kernel,
        mesh=mesh,
        out_type=jax.ShapeDtypeStruct((B_PAD, D_FEAT), jnp.float32),
        scratch_types=[
            pltpu.VMEM((64,), jnp.int32),
            pltpu.VMEM((64, D_FEAT), jnp.float32),
            pltpu.SemaphoreType.DMA,
        ],
    )
    def k(h_h, nodes_h, out_o, idx_v, rows_v, sem):
        c = lax.axis_index("c")
        s = lax.axis_index("s")
        wid = s * NC + c
        base = pl.multiple_of(wid * BPT, 64)

        @pl.loop(0, BPT // 64)
        def _g(j):
            off = pl.multiple_of(base + j * 64, 64)
            pltpu.sync_copy(nodes_h.at[pl.ds(off, 64)], idx_v)
            pltpu.async_copy(h_h.at[idx_v], rows_v, sem).wait()
            pltpu.sync_copy(rows_v, out_o.at[pl.ds(off, 64)])

    return k(h, nodesp)


def _dense_tc(x, n0, n1, d0, d1, w1t, w2at, w2bt):
    BLK = 1000
    grid = (N_NODES // BLK,)

    def mm(x_r, n0_r, n1_r, d0_r, d1_r, w1_r, w2a_r, w2b_r, o_r):
        deg = d0_r[:, 0:1] + d1_r[:, 0:1]
        inv = 1.0 / jnp.maximum(deg, 1.0)
        acc = jnp.dot(x_r[...], w1_r[...], preferred_element_type=jnp.float32)
        acc += jnp.dot(n0_r[...] * inv, w2a_r[...], preferred_element_type=jnp.float32)
        acc += jnp.dot(n1_r[...] * inv, w2b_r[...], preferred_element_type=jnp.float32)
        o_r[...] = jnp.maximum(acc, 0.0)

    return pl.pallas_call(
        mm,
        grid=grid,
        in_specs=[
            pl.BlockSpec((BLK, D_FEAT), lambda i: (i, 0)),
            pl.BlockSpec((BLK, HALF), lambda i: (i, 0)),
            pl.BlockSpec((BLK, HALF), lambda i: (i, 0)),
            pl.BlockSpec((BLK, HALF), lambda i: (i, 0)),
            pl.BlockSpec((BLK, HALF), lambda i: (i, 0)),
            pl.BlockSpec((D_FEAT, D_FEAT), lambda i: (0, 0)),
            pl.BlockSpec((HALF, D_FEAT), lambda i: (0, 0)),
            pl.BlockSpec((HALF, D_FEAT), lambda i: (0, 0)),
        ],
        out_specs=pl.BlockSpec((BLK, D_FEAT), lambda i: (i, 0)),
        out_shape=jax.ShapeDtypeStruct((N_NODES, D_FEAT), jnp.float32),
    )(x, n0, n1, d0, d1, w1t, w2at, w2bt)


def kernel(nodes, x, edge_index, W):
    src = edge_index[0]
    dst = edge_index[1]

    xr = x.reshape(N_NODES * NC, HALF)
    pad = E_PAD - N_EDGES
    srcp = jnp.concatenate([src, jnp.zeros((pad,), jnp.int32)])
    dstp = jnp.concatenate([dst, jnp.full((pad,), N_NODES, jnp.int32)])
    idx2f = jnp.concatenate([srcp * 2, srcp * 2 + 1])
    zf = jnp.zeros((128, HALF), jnp.float32)
    zd = jnp.zeros((128, 128), jnp.float32)
    ones_f = jnp.ones((128, 128), jnp.float32)

    neigh2 = _seg_sum_sc(xr, idx2f, dstp, zf)
    degs = _deg_sc(dstp, zd, ones_f)

    n0 = neigh2[:N_NODES]
    n1 = neigh2[N_PAD:N_PAD + N_NODES]
    d0 = degs[:N_NODES]
    d1 = degs[N_PAD:N_PAD + N_NODES]
    w1t = W[:, :D_FEAT].T
    w2at = W[:, D_FEAT:D_FEAT + HALF].T
    w2bt = W[:, D_FEAT + HALF:].T

    h = _dense_tc(x, n0, n1, d0, d1, w1t, w2at, w2bt)

    nodesp = jnp.concatenate([nodes, jnp.zeros((B_PAD - N_NODES,), jnp.int32)])
    g = _gather_sc(h, nodesp)
    return g[:N_NODES]


# speedup vs baseline: 2.9265x; 1.1333x over previous
import functools

import jax
import jax.numpy as jnp
from jax import lax
from jax.experimental import pallas as pl
from jax.experimental.pallas import tpu as pltpu
from jax.experimental.pallas import tpu_sc as plsc

N_NODES = 10000
N_EDGES = 160000
D_FEAT = 256
HALF = 128

NC = 2
NS = 16
NW = NC * NS

E_PAD = 163840
EPT = E_PAD // NS
FULL_CHUNKS = EPT // 128
DEG_CHUNKS = FULL_CHUNKS // NC
N_PAD = 10112
ROW0_STRIDE = 640
D_PAD = 10240

B_PAD = 10240
BPT = B_PAD // NW


def _seg_sum_sc(xr, idx2r, dst2r, zf, zi):
    mesh = plsc.VectorSubcoreMesh(core_axis_name="c", subcore_axis_name="s")

    @functools.partial(
        pl.kernel,
        mesh=mesh,
        out_type=jax.ShapeDtypeStruct((NC * N_PAD, HALF), jnp.float32),
        scratch_types=[
            pltpu.VMEM((FULL_CHUNKS + 8, 1, 128), jnp.int32),
            pltpu.VMEM((FULL_CHUNKS, 1, 128), jnp.int32),
            pltpu.VMEM((128, HALF), jnp.float32),
            pltpu.VMEM_SHARED((N_PAD, HALF), jnp.float32),
            pltpu.SemaphoreType.DMA,
        ],
    )
    def k(xr_h, idx2_h, dst_h, zf_h, zi_h,
          neigh_o,
          idx_a, dst_a, rows_v, acc, sem):
        c = lax.axis_index("c")
        s = lax.axis_index("s")
        row0 = pl.multiple_of(s * ROW0_STRIDE, 128)
        nck = jnp.where(s == NS - 1, 4, 5)

        pltpu.sync_copy(zf_h, rows_v)

        @pl.loop(0, nck)
        def _zero(kk):
            roff = pl.multiple_of(row0 + kk * 128, 128)
            pltpu.sync_copy(rows_v, acc.at[pl.ds(roff, 128)])

        ibase = pl.multiple_of((c * NS + s) * FULL_CHUNKS, 8)
        dbase = pl.multiple_of(s * FULL_CHUNKS, 8)
        pltpu.sync_copy(idx2_h.at[pl.ds(ibase, FULL_CHUNKS)], idx_a.at[pl.ds(0, FULL_CHUNKS)])
        pltpu.sync_copy(zi_h, idx_a.at[pl.ds(FULL_CHUNKS, 8)])
        pltpu.sync_copy(dst_h.at[pl.ds(dbase, FULL_CHUNKS)], dst_a)
        plsc.subcore_barrier()

        @pl.loop(0, FULL_CHUNKS)
        def _feat(j):
            pltpu.async_copy(xr_h.at[idx_a.at[j, 0]], rows_v, sem).wait()
            pltpu.sync_copy(rows_v, acc.at[dst_a.at[j, 0]], add=True)

        plsc.subcore_barrier()

        orow0 = pl.multiple_of(c * N_PAD + s * ROW0_STRIDE, 128)

        @pl.loop(0, nck)
        def _wb(kk):
            roff = pl.multiple_of(row0 + kk * 128, 128)
            ooff = pl.multiple_of(orow0 + kk * 128, 128)
            pltpu.sync_copy(acc.at[pl.ds(roff, 128)], rows_v)
            pltpu.sync_copy(rows_v, neigh_o.at[pl.ds(ooff, 128)])

    return k(xr, idx2r, dst2r, zf, zi)


def _deg_sc(dst2r, zd, ones_f):
    mesh = plsc.VectorSubcoreMesh(core_axis_name="c", subcore_axis_name="s")
    DCK = E_PAD // (NC * NS * 128)

    @functools.partial(
        pl.kernel,
        mesh=mesh,
        out_type=jax.ShapeDtypeStruct((NC * N_PAD, 128), jnp.float32),
        scratch_types=[
            pltpu.VMEM((DCK, 1, 128), jnp.int32),
            pltpu.VMEM((128, 128), jnp.float32),
            pltpu.VMEM((128, 128), jnp.float32),
            pltpu.VMEM_SHARED((N_PAD, 128), jnp.float32),
        ],
    )
    def k(dst_h, zd_h, ones_fh, degs_o, dst_a, ones_v, small_v, dacc):
        c = lax.axis_index("c")
        s = lax.axis_index("s")
        row0 = pl.multiple_of(s * ROW0_STRIDE, 128)
        nck = jnp.where(s == NS - 1, 4, 5)

        pltpu.sync_copy(zd_h, small_v)
        pltpu.sync_copy(ones_fh, ones_v)
        gbase = pl.multiple_of((c * NS + s) * DCK, 8)
        pltpu.sync_copy(dst_h.at[pl.ds(gbase, DCK)], dst_a)

        @pl.loop(0, nck)
        def _zero(kk):
            roff = pl.multiple_of(row0 + kk * 128, 128)
            pltpu.sync_copy(small_v, dacc.at[pl.ds(roff, 128)])

        plsc.subcore_barrier()

        @pl.loop(0, DCK)
        def _deg(j):
            pltpu.sync_copy(ones_v, dacc.at[dst_a.at[j, 0]], add=True)

        plsc.subcore_barrier()

        orow0 = pl.multiple_of(c * N_PAD + s * ROW0_STRIDE, 128)

        @pl.loop(0, nck)
        def _wb(kk):
            roff = pl.multiple_of(row0 + kk * 128, 128)
            ooff = pl.multiple_of(orow0 + kk * 128, 128)
            pltpu.sync_copy(dacc.at[pl.ds(roff, 128)], small_v)
            pltpu.sync_copy(small_v, degs_o.at[pl.ds(ooff, 128)])

    return k(dst2r, zd, ones_f)


def _gather_sc(h, nodesr):
    mesh = plsc.VectorSubcoreMesh(core_axis_name="c", subcore_axis_name="s")
    NCK = BPT // 64

    @functools.partial(
        pl.kernel,
        mesh=mesh,
        out_type=jax.ShapeDtypeStruct((B_PAD, D_FEAT), jnp.float32),
        scratch_types=[
            pltpu.VMEM((NCK, 1, 64), jnp.int32),
            pltpu.VMEM((64, D_FEAT), jnp.float32),
            pltpu.VMEM((64, D_FEAT), jnp.float32),
            pltpu.SemaphoreType.DMA,
            pltpu.SemaphoreType.DMA,
        ],
    )
    def k(h_h, nodes_h, out_o, idx_a, rows0, rows1, sem0, sem1):
        c = lax.axis_index("c")
        s = lax.axis_index("s")
        wid = s * NC + c
        base = pl.multiple_of(wid * NCK, 8)
        pltpu.sync_copy(nodes_h.at[pl.ds(base, NCK)], idx_a)

        bufs = ((rows0, sem0), (rows1, sem1))
        pltpu.async_copy(h_h.at[idx_a.at[0, 0]], rows0, sem0)
        for t in range(NCK):
            rb, sb = bufs[t % 2]
            if t + 1 < NCK:
                rn, sn = bufs[(t + 1) % 2]
                pltpu.async_copy(h_h.at[idx_a.at[t + 1, 0]], rn, sn)
            pltpu.make_async_copy(h_h.at[idx_a.at[t, 0]], rb, sb).wait()
            off = pl.multiple_of(wid * BPT + t * 64, 64)
            pltpu.sync_copy(rb, out_o.at[pl.ds(off, 64)])

    return k(h, nodesr)


def _dense_tc(x, n0, n1, d0, d1, w1t, w2at, w2bt):
    BLK = 1000
    grid = (N_NODES // BLK,)

    def mm(x_r, n0_r, n1_r, d0_r, d1_r, w1_r, w2a_r, w2b_r, o_r):
        deg = d0_r[:, 0:1] + d1_r[:, 0:1]
        inv = 1.0 / jnp.maximum(deg, 1.0)
        acc = jnp.dot(x_r[...], w1_r[...], preferred_element_type=jnp.float32)
        acc += jnp.dot(n0_r[...] * inv, w2a_r[...], preferred_element_type=jnp.float32)
        acc += jnp.dot(n1_r[...] * inv, w2b_r[...], preferred_element_type=jnp.float32)
        o_r[...] = jnp.maximum(acc, 0.0)

    return pl.pallas_call(
        mm,
        grid=grid,
        in_specs=[
            pl.BlockSpec((BLK, D_FEAT), lambda i: (i, 0)),
            pl.BlockSpec((BLK, HALF), lambda i: (i, 0)),
            pl.BlockSpec((BLK, HALF), lambda i: (i, 0)),
            pl.BlockSpec((BLK, HALF), lambda i: (i, 0)),
            pl.BlockSpec((BLK, HALF), lambda i: (i, 0)),
            pl.BlockSpec((D_FEAT, D_FEAT), lambda i: (0, 0)),
            pl.BlockSpec((HALF, D_FEAT), lambda i: (0, 0)),
            pl.BlockSpec((HALF, D_FEAT), lambda i: (0, 0)),
        ],
        out_specs=pl.BlockSpec((BLK, D_FEAT), lambda i: (i, 0)),
        out_shape=jax.ShapeDtypeStruct((N_NODES, D_FEAT), jnp.float32),
    )(x, n0, n1, d0, d1, w1t, w2at, w2bt)


def kernel(nodes, x, edge_index, W):
    src = edge_index[0]
    dst = edge_index[1]

    xr = x.reshape(N_NODES * NC, HALF)
    pad = E_PAD - N_EDGES
    srcp = jnp.concatenate([src, jnp.zeros((pad,), jnp.int32)])
    dstp = jnp.concatenate([dst, jnp.full((pad,), N_NODES, jnp.int32)])
    idx2r = jnp.concatenate([srcp * 2, srcp * 2 + 1]).reshape(-1, 1, 128)
    dst2r = dstp.reshape(-1, 1, 128)
    zf = jnp.zeros((128, HALF), jnp.float32)
    zi = jnp.zeros((8, 1, 128), jnp.int32)
    zd = jnp.zeros((128, 128), jnp.float32)
    ones_f = jnp.ones((128, 128), jnp.float32)

    neigh2 = _seg_sum_sc(xr, idx2r, dst2r, zf, zi)
    degs = _deg_sc(dst2r, zd, ones_f)

    n0 = neigh2[:N_NODES]
    n1 = neigh2[N_PAD:N_PAD + N_NODES]
    d0 = degs[:N_NODES]
    d1 = degs[N_PAD:N_PAD + N_NODES]
    w1t = W[:, :D_FEAT].T
    w2at = W[:, D_FEAT:D_FEAT + HALF].T
    w2bt = W[:, D_FEAT + HALF:].T

    h = _dense_tc(x, n0, n1, d0, d1, w1t, w2at, w2bt)

    nodesp = jnp.concatenate([nodes, jnp.zeros((B_PAD - N_NODES,), jnp.int32)])
    g = _gather_sc(h, nodesp.reshape(-1, 1, 64))
    return g[:N_NODES]
